# baseline (speedup 1.0000x reference)
import jax
import jax.numpy as jnp
from jax import lax
from jax.experimental import pallas as pl
from jax.experimental.pallas import tpu as pltpu
from jax.experimental.pallas import tpu_sc as plsc

_D = 128
_E = 16
_H = 16
_LANES = 16
_CHUNKS = 2
_NW = 32
_SC_S = 512
_TC_SBLK = 512


def _tc_router_body(x_ref, w_ref, o_ref):
    sb, h, d = x_ref.shape[1], x_ref.shape[2], x_ref.shape[3]
    e = w_ref.shape[1]
    x2 = x_ref[...].reshape(sb * h, d)
    logits = jnp.dot(x2, w_ref[...], preferred_element_type=jnp.float32)
    m = jnp.max(logits, axis=-1, keepdims=True)
    ex = jnp.exp(logits - m)
    res = ex / jnp.sum(ex, axis=-1, keepdims=True)
    o_ref[...] = res.reshape(1, sb, h, e)


def _make_sc_body(s_base, toks_per_w):
    def _sc_router_body(x_hbm, ws_hbm, o_hbm, ws_v, xb_v, ob_v):
        nb = _NW // 2
        wid = lax.axis_index("s") * 2 + lax.axis_index("c")
        b = wid // nb
        tok0 = s_base + (wid % nb) * toks_per_w
        otok0 = (wid % nb) * toks_per_w
        pltpu.sync_copy(ws_hbm, ws_v)
        lanes = jnp.arange(_LANES, dtype=jnp.int32)

        _TS = 16
        for st in range(toks_per_w // _TS):
          pltpu.sync_copy(x_hbm.at[b, pl.ds(tok0 + st * _TS, _TS)], xb_v)
          for blk in range(_TS // _CHUNKS):
            toks = [blk * _CHUNKS + c for c in range(_CHUNKS)]

            def dbody(d, accs):
                col = jnp.full((_LANES,), d, dtype=jnp.int32)
                xT = [
                    plsc.load_gather(
                        xb_v,
                        [jnp.full((_LANES,), t, dtype=jnp.int32), lanes, col])
                    for t in toks
                ]
                return tuple(
                    accs[c * _E + e] + xT[c] * ws_v[d, pl.ds(e * _LANES, _LANES)]
                    for c in range(_CHUNKS)
                    for e in range(_E)
                )

            accs = lax.fori_loop(
                0, _D, dbody,
                tuple(jnp.zeros((_LANES,), jnp.float32)
                      for _ in range(_CHUNKS * _E)),
            )
            for c in range(_CHUNKS):
                ac = accs[c * _E:(c + 1) * _E]
                m = ac[0]
                for e in range(1, _E):
                    m = jnp.maximum(m, ac[e])
                es = [jnp.exp(a - m) for a in ac]
                s = es[0]
                for e in range(1, _E):
                    s = s + es[e]
                r = 1.0 / s
                tvec = jnp.full((_LANES,), toks[c], dtype=jnp.int32)
                for e in range(_E):
                    plsc.store_scatter(
                        ob_v,
                        [tvec, lanes, jnp.full((_LANES,), e, dtype=jnp.int32)],
                        es[e] * r)
          pltpu.sync_copy(
              ob_v, o_hbm.at[b, pl.ds(otok0 + st * _TS, _TS)])

    return _sc_router_body


def _sc_router(x, wsplat, s_base):
    B = x.shape[0]
    toks_per_w = (_SC_S * B) // _NW
    mesh = plsc.VectorSubcoreMesh(core_axis_name="c", subcore_axis_name="s")
    f = pl.kernel(
        _make_sc_body(s_base, toks_per_w),
        mesh=mesh,
        out_type=jax.ShapeDtypeStruct((B, _SC_S, _H, _E), jnp.float32),
        compiler_params=pltpu.CompilerParams(
            needs_layout_passes=False, use_tc_tiling_on_sc=True),
        scratch_types=[
            pltpu.VMEM((_D, _E * _LANES), jnp.float32),
            pltpu.VMEM((16, _H, _D), jnp.float32),
            pltpu.VMEM((16, _H, _E), jnp.float32),
        ],
    )
    return f(x, wsplat)


def kernel(x, expert_embeddings):
    B, S, H, D = x.shape
    E = expert_embeddings.shape[1]
    s_tc = S - _SC_S

    wsplat = jnp.broadcast_to(
        expert_embeddings.reshape(D, E, 1), (D, E, _LANES)
    ).reshape(D, E * _LANES)
    x_tail = lax.slice(
        x, (0, s_tc, 0, 0), (B, S, H, D))
    out_sc = _sc_router(x_tail, wsplat, 0)

    out_tc = pl.pallas_call(
        _tc_router_body,
        grid=(B, s_tc // _TC_SBLK),
        in_specs=[
            pl.BlockSpec((1, _TC_SBLK, H, D), lambda b, s: (b, s, 0, 0)),
            pl.BlockSpec((D, E), lambda b, s: (0, 0)),
        ],
        out_specs=pl.BlockSpec((1, _TC_SBLK, H, E), lambda b, s: (b, s, 0, 0)),
        out_shape=jax.ShapeDtypeStruct((B, S, H, E), jnp.float32),
    )(x, expert_embeddings)

    return lax.dynamic_update_slice(out_tc, out_sc, (0, s_tc, 0, 0))


# speedup vs baseline: 1.7786x; 1.7786x over previous
import jax
import jax.numpy as jnp
from jax.experimental import pallas as pl

_SBLK = 1024


def _tc_router_body(x_ref, w_ref, o_ref):
    sb, h, d = x_ref.shape[1], x_ref.shape[2], x_ref.shape[3]
    e = w_ref.shape[1]
    x2 = x_ref[...].reshape(sb * h, d)
    logits = jnp.dot(x2, w_ref[...], preferred_element_type=jnp.float32)
    m = jnp.max(logits, axis=-1, keepdims=True)
    ex = jnp.exp(logits - m)
    res = ex / jnp.sum(ex, axis=-1, keepdims=True)
    o_ref[...] = res.reshape(1, sb, h, e)


def kernel(x, expert_embeddings):
    B, S, H, D = x.shape
    E = expert_embeddings.shape[1]
    return pl.pallas_call(
        _tc_router_body,
        grid=(B, S // _SBLK),
        in_specs=[
            pl.BlockSpec((1, _SBLK, H, D), lambda b, s: (b, s, 0, 0)),
            pl.BlockSpec((D, E), lambda b, s: (0, 0)),
        ],
        out_specs=pl.BlockSpec((1, _SBLK, H, E), lambda b, s: (b, s, 0, 0)),
        out_shape=jax.ShapeDtypeStruct((B, S, H, E), jnp.float32),
    )(x, expert_embeddings)
